# baseline (device time: 34391 ns/iter reference)
import os

import jax
import jax.numpy as jnp
from jax import lax
from jax.experimental import pallas as pl
from jax.experimental.pallas import tpu as pltpu

N_DEV = 16
NSB = 4
E4M3_MAX = 448.0
VARIANT = os.environ.get("KVARIANT", "full")


def _quant_dequant_e4m3(z, inv_scale, scale):
    t = z * inv_scale
    t = jnp.clip(t, -E4M3_MAX, E4M3_MAX)
    u = lax.bitcast_convert_type(t, jnp.uint32)
    u = (u + jnp.uint32(0x7FFFF) + ((u >> jnp.uint32(20)) & jnp.uint32(1))) & jnp.uint32(
        0xFFF00000
    )
    t = lax.bitcast_convert_type(u, jnp.float32)
    return t * scale


def kernel(x, w_mat):
    m_per, k = x.shape
    _, n = w_mat.shape
    n_per = n // N_DEV
    n_sb = n // NSB
    per_sb = n_sb // n_per

    def body(
        x_ref,
        w_ref,
        out_ref,
        wbuf,
        y_src,
        recv,
        amax_src,
        amax_recv,
        wsem,
        dsend,
        drecv,
        asend,
        arecv,
    ):
        my = lax.axis_index("i")

        def w_copy(s):
            return pltpu.make_async_copy(
                w_ref.at[:, s * n_sb : (s + 1) * n_sb],
                wbuf.at[s % 2],
                wsem.at[s % 2],
            )

        w_copy(0).start()

        xb = x_ref[...].astype(jnp.bfloat16)

        data_rdmas = []
        vmax = None
        for s in range(NSB):
            if s + 1 < NSB:
                w_copy(s + 1).start()
            w_copy(s).wait()
            wb = wbuf[s % 2].astype(jnp.bfloat16)
            yb = jnp.dot(xb, wb, preferred_element_type=jnp.float32)
            pmax = jnp.max(jnp.abs(yb), axis=0, keepdims=True)
            vmax = pmax if vmax is None else jnp.maximum(vmax, pmax)
            y_src[:, s * n_sb : (s + 1) * n_sb] = yb.astype(jnp.bfloat16)
            if VARIANT == "nocomm":
                continue
            for i in range(per_sb):
                j = s * per_sb + i
                c = pltpu.make_async_remote_copy(
                    src_ref=y_src.at[:, pl.ds(j * n_per, n_per)],
                    dst_ref=recv.at[pl.ds(my * m_per, m_per), :],
                    send_sem=dsend.at[j],
                    recv_sem=drecv.at[my],
                    device_id=(j,),
                    device_id_type=pl.DeviceIdType.MESH,
                )

                @pl.when(my != j)
                def _(c=c):
                    c.start()

                data_rdmas.append((j, c))

        recv[pl.ds(my * m_per, m_per), :] = y_src[:, pl.ds(my * n_per, n_per)]

        v = jnp.max(vmax.reshape(n_sb // n_per, n_per), axis=0, keepdims=True)
        amax_src[...] = jnp.zeros((8, 128), jnp.float32) + v
        amax_recv[0, :, :] = jnp.zeros((8, 128), jnp.float32) + v

        amax_rdmas = []
        for d in range(1, N_DEV) if VARIANT != "nocomm" else []:
            jd = lax.rem(my + d, N_DEV)
            a = pltpu.make_async_remote_copy(
                src_ref=amax_src,
                dst_ref=amax_recv.at[d],
                send_sem=asend.at[d],
                recv_sem=arecv.at[d],
                device_id=(jd,),
                device_id_type=pl.DeviceIdType.MESH,
            )
            a.start()
            amax_rdmas.append(a)

        for a in amax_rdmas:
            a.wait_recv()
        gmax = jnp.max(amax_recv[...])
        scale = gmax / E4M3_MAX
        inv_scale = E4M3_MAX / gmax

        if VARIANT != "nocomm":
            for d in range(1, N_DEV):
                i = lax.rem(my + d, N_DEV)
                rwait = pltpu.make_async_remote_copy(
                    src_ref=y_src.at[:, pl.ds(0, n_per)],
                    dst_ref=recv.at[pl.ds(i * m_per, m_per), :],
                    send_sem=dsend.at[0],
                    recv_sem=drecv.at[i],
                    device_id=(0,),
                    device_id_type=pl.DeviceIdType.MESH,
                )
                rwait.wait_recv()

        out_ref[...] = _quant_dequant_e4m3(
            recv[...].astype(jnp.float32), inv_scale, scale
        )

        for j, c in data_rdmas:
            @pl.when(my != j)
            def _(c=c):
                c.wait_send()
        for a in amax_rdmas:
            a.wait_send()

    return pl.pallas_call(
        body,
        out_shape=jax.ShapeDtypeStruct((N_DEV * m_per, n_per), jnp.float32),
        in_specs=[
            pl.BlockSpec(memory_space=pltpu.VMEM),
            pl.BlockSpec(memory_space=pltpu.MemorySpace.HBM),
        ],
        out_specs=pl.BlockSpec(memory_space=pltpu.VMEM),
        scratch_shapes=[
            pltpu.VMEM((2, k, n_sb), jnp.float32),
            pltpu.VMEM((m_per, n), jnp.bfloat16),
            pltpu.VMEM((N_DEV * m_per, n_per), jnp.bfloat16),
            pltpu.VMEM((8, 128), jnp.float32),
            pltpu.VMEM((N_DEV, 8, 128), jnp.float32),
            pltpu.SemaphoreType.DMA((2,)),
            pltpu.SemaphoreType.DMA((N_DEV,)),
            pltpu.SemaphoreType.DMA((N_DEV,)),
            pltpu.SemaphoreType.DMA((N_DEV,)),
            pltpu.SemaphoreType.DMA((N_DEV,)),
        ],
    )(x, w_mat)
